# Initial kernel scaffold; baseline (speedup 1.0000x reference)
import jax
import jax.numpy as jnp
from jax.experimental import pallas as pl


def kernel(e, edge_index):
    raise NotImplementedError("write your pallas kernel here")


# speedup vs baseline: 96.2377x; 96.2377x over previous
import functools

import jax
import jax.numpy as jnp
from jax import lax
from jax.experimental import pallas as pl
from jax.experimental.pallas import tpu as pltpu
from jax.experimental.pallas import tpu_sc as plsc

NUM_NODES = 100000
NUM_EDGES = 6400000

NC = 2
NS = 16
NW = NC * NS
L = 16

EPT = NUM_EDGES // NW
EBLK = 4000
NBLK = EPT // EBLK
NPAD = 102400
NSLICE = NPAD // NS

def _mesh():
    return plsc.VectorSubcoreMesh(
        core_axis_name="c",
        subcore_axis_name="s",
        num_cores=NC,
        num_subcores=NS,
    )


def _phase1(e_hbm, ei_hbm, part_hbm, table_v, e_v, idx_v, chunk_v, shared_v):
    cid = lax.axis_index("c")
    sid = lax.axis_index("s")
    wid = cid * NS + sid

    def zero_body(i, _):
        table_v[pl.ds(i * L, L)] = jnp.zeros((L,), jnp.float32)
        return 0

    lax.fori_loop(0, NPAD // L, zero_body, 0)

    def edge_body(i, _):
        idx = idx_v[pl.ds(i * L, L)]
        ex = jnp.exp(e_v[pl.ds(i * L, L)])
        plsc.addupdate_scatter(table_v, [idx], ex)
        return 0

    def block_body(b, _):
        base = wid * EPT + b * EBLK
        pltpu.sync_copy(e_hbm.at[pl.ds(base, EBLK)], e_v)
        pltpu.sync_copy(ei_hbm.at[pl.ds(NUM_EDGES + base, EBLK)], idx_v)
        lax.fori_loop(0, EBLK // L, edge_body, 0)
        return 0

    lax.fori_loop(0, NBLK, block_body, 0)

    def add_body(i, _):
        s = pl.ds(sid * NSLICE + i * L, L)
        table_v[s] = table_v[s] + chunk_v[pl.ds(i * L, L)]
        return 0

    def round_body(r, _):
        o = lax.rem(sid + r, NS)
        pltpu.sync_copy(table_v.at[pl.ds(o * NSLICE, NSLICE)], shared_v.at[o])
        plsc.subcore_barrier()
        pltpu.sync_copy(shared_v.at[sid], chunk_v)
        lax.fori_loop(0, NSLICE // L, add_body, 0)
        plsc.subcore_barrier()
        return 0

    lax.fori_loop(1, NS, round_body, 0)
    pltpu.sync_copy(table_v.at[pl.ds(sid * NSLICE, NSLICE)],
                    part_hbm.at[cid, pl.ds(sid * NSLICE, NSLICE)])


def _phase2(e_hbm, ei_hbm, part_hbm, alpha_hbm, table_v, e_v, idx_v, out_v,
            chunk_v):
    cid = lax.axis_index("c")
    sid = lax.axis_index("s")
    wid = cid * NS + sid

    pltpu.sync_copy(part_hbm.at[0], table_v)

    def add_body(c, _):
        def lane_body(i, _):
            s = pl.ds(c * NSLICE + i * L, L)
            table_v[s] = table_v[s] + chunk_v[pl.ds(i * L, L)]
            return 0

        pltpu.sync_copy(part_hbm.at[1, pl.ds(c * NSLICE, NSLICE)], chunk_v)
        lax.fori_loop(0, NSLICE // L, lane_body, 0)
        return 0

    lax.fori_loop(0, NS, add_body, 0)

    def edge_body(i, _):
        s = pl.ds(i * L, L)
        idx = idx_v[s]
        g = plsc.load_gather(table_v, [idx])
        out_v[s] = jnp.exp(e_v[s]) / (g + 1e-16)
        return 0

    def block_body(b, _):
        base = wid * EPT + b * EBLK
        pltpu.sync_copy(e_hbm.at[pl.ds(base, EBLK)], e_v)
        pltpu.sync_copy(ei_hbm.at[pl.ds(NUM_EDGES + base, EBLK)], idx_v)
        lax.fori_loop(0, EBLK // L, edge_body, 0)
        pltpu.sync_copy(out_v, alpha_hbm.at[pl.ds(base, EBLK)])
        return 0

    lax.fori_loop(0, NBLK, block_body, 0)


@functools.lru_cache(maxsize=None)
def _build():
    phase1 = pl.kernel(
        _phase1,
        out_type=jax.ShapeDtypeStruct((NC, NPAD), jnp.float32),
        mesh=_mesh(),
        compiler_params=pltpu.CompilerParams(needs_layout_passes=False),
        scratch_types=[
            pltpu.VMEM((NPAD,), jnp.float32),
            pltpu.VMEM((EBLK,), jnp.float32),
            pltpu.VMEM((EBLK,), jnp.int32),
            pltpu.VMEM((NSLICE,), jnp.float32),
            pltpu.VMEM_SHARED((NS, NSLICE), jnp.float32),
        ],
    )
    phase2 = pl.kernel(
        _phase2,
        out_type=jax.ShapeDtypeStruct((NUM_EDGES,), jnp.float32),
        mesh=_mesh(),
        compiler_params=pltpu.CompilerParams(needs_layout_passes=False),
        scratch_types=[
            pltpu.VMEM((NPAD,), jnp.float32),
            pltpu.VMEM((EBLK,), jnp.float32),
            pltpu.VMEM((EBLK,), jnp.int32),
            pltpu.VMEM((EBLK,), jnp.float32),
            pltpu.VMEM((NSLICE,), jnp.float32),
        ],
    )
    return phase1, phase2


@jax.jit
def kernel(e, edge_index):
    phase1, phase2 = _build()
    dst_flat = edge_index.astype(jnp.int32).reshape(-1)
    partials = phase1(e, dst_flat)
    return phase2(e, dst_flat, partials)


# speedup vs baseline: 141.0825x; 1.4660x over previous
import functools

import jax
import jax.numpy as jnp
from jax import lax
from jax.experimental import pallas as pl
from jax.experimental.pallas import tpu as pltpu
from jax.experimental.pallas import tpu_sc as plsc

NUM_NODES = 100000
NUM_EDGES = 6400000

NC = 2
NS = 16
NW = NC * NS
L = 16

EPT = NUM_EDGES // NW
EBLK = 4000
NBLK = EPT // EBLK
U = 5
NPAD = 102400
NSLICE = NPAD // NS
CCH = 3200


def _mesh():
    return plsc.VectorSubcoreMesh(
        core_axis_name="c",
        subcore_axis_name="s",
        num_cores=NC,
        num_subcores=NS,
    )


def _start(hbm, off, n, buf, sem):
    pltpu.make_async_copy(hbm.at[pl.ds(off, n)], buf, sem).start()


def _wait(hbm, off, n, buf, sem):
    pltpu.make_async_copy(hbm.at[pl.ds(off, n)], buf, sem).wait()


def _phase1(e_hbm, ei_hbm, part_hbm, table_v, e_v0, e_v1, i_v0, i_v1,
            shared_v, se0, se1, si0, si1):
    cid = lax.axis_index("c")
    sid = lax.axis_index("s")
    wid = cid * NS + sid
    ebase = wid * EPT

    def zero_body(i, _):
        for u in range(8):
            table_v[pl.ds((i * 8 + u) * L, L)] = jnp.zeros((L,), jnp.float32)
        return 0

    lax.fori_loop(0, NPAD // (8 * L), zero_body, 0)

    def scat_chunk(eb, ib):
        def edge_body(i, _):
            for u in range(U):
                s = pl.ds((i * U + u) * L, L)
                plsc.addupdate_scatter(table_v, [ib[s]], jnp.exp(eb[s]))
            return 0

        lax.fori_loop(0, EBLK // (L * U), edge_body, 0)

    _start(e_hbm, ebase, EBLK, e_v0, se0)
    _start(ei_hbm, NUM_EDGES + ebase, EBLK, i_v0, si0)

    def dbl_body(g, _):
        b1 = ebase + (2 * g + 1) * EBLK
        _start(e_hbm, b1, EBLK, e_v1, se1)
        _start(ei_hbm, NUM_EDGES + b1, EBLK, i_v1, si1)
        _wait(e_hbm, b1, EBLK, e_v0, se0)
        _wait(ei_hbm, b1, EBLK, i_v0, si0)
        scat_chunk(e_v0, i_v0)

        @pl.when(g + 1 < NBLK // 2)
        def _():
            b2 = ebase + (2 * g + 2) * EBLK
            _start(e_hbm, b2, EBLK, e_v0, se0)
            _start(ei_hbm, NUM_EDGES + b2, EBLK, i_v0, si0)

        _wait(e_hbm, b1, EBLK, e_v1, se1)
        _wait(ei_hbm, b1, EBLK, i_v1, si1)
        scat_chunk(e_v1, i_v1)
        return 0

    lax.fori_loop(0, NBLK // 2, dbl_body, 0)

    H = NSLICE // 2

    def add_half(h, buf):
        def add_body(i, _):
            for u in range(8):
                off = (i * 8 + u) * L
                s = pl.ds(sid * NSLICE + h * H + off, L)
                table_v[s] = table_v[s] + buf[pl.ds(off, L)]
            return 0

        lax.fori_loop(0, H // (8 * L), add_body, 0)

    def round_body(r, _):
        o = lax.rem(sid + r, NS)
        pltpu.sync_copy(table_v.at[pl.ds(o * NSLICE, NSLICE)], shared_v.at[o])
        plsc.subcore_barrier()
        pltpu.sync_copy(shared_v.at[sid, pl.ds(0, H)], e_v0.at[pl.ds(0, H)])
        pltpu.sync_copy(shared_v.at[sid, pl.ds(H, H)], e_v1.at[pl.ds(0, H)])
        add_half(0, e_v0)
        add_half(1, e_v1)
        plsc.subcore_barrier()
        return 0

    lax.fori_loop(1, NS, round_body, 0)
    pltpu.sync_copy(table_v.at[pl.ds(sid * NSLICE, NSLICE)],
                    part_hbm.at[cid, pl.ds(sid * NSLICE, NSLICE)])


def _phase2(e_hbm, ei_hbm, part_hbm, alpha_hbm, table_v, e_v0, e_v1, i_v0,
            i_v1, o_v0, o_v1, se0, se1, si0, si1, so0, so1):
    cid = lax.axis_index("c")
    sid = lax.axis_index("s")
    wid = cid * NS + sid
    ebase = wid * EPT

    pltpu.sync_copy(part_hbm.at[0], table_v)

    def comb_chunk(c, buf):
        def add_body(i, _):
            for u in range(8):
                s = pl.ds(c * CCH + (i * 8 + u) * L, L)
                table_v[s] = table_v[s] + buf[pl.ds((i * 8 + u) * L, L)]
            return 0

        lax.fori_loop(0, CCH // (8 * L), add_body, 0)

    def _startp(c, buf, sem):
        pltpu.make_async_copy(part_hbm.at[1, pl.ds(c * CCH, CCH)],
                              buf.at[pl.ds(0, CCH)], sem).start()

    def _waitp(buf, sem):
        pltpu.make_async_copy(part_hbm.at[1, pl.ds(0, CCH)],
                              buf.at[pl.ds(0, CCH)], sem).wait()

    NCH = NPAD // CCH
    _startp(0, e_v0, se0)

    def comb_body(g, _):
        c0 = 2 * g
        _startp(c0 + 1, e_v1, se1)
        _waitp(e_v0, se0)
        comb_chunk(c0, e_v0)

        @pl.when(c0 + 2 < NCH)
        def _():
            _startp(c0 + 2, e_v0, se0)

        _waitp(e_v1, se1)
        comb_chunk(c0 + 1, e_v1)
        return 0

    lax.fori_loop(0, NCH // 2, comb_body, 0)

    def gath_chunk(eb, ib, ob):
        def edge_body(i, _):
            for u in range(U):
                s = pl.ds((i * U + u) * L, L)
                g = plsc.load_gather(table_v, [ib[s]])
                ob[s] = jnp.exp(eb[s]) / (g + 1e-16)
            return 0

        lax.fori_loop(0, EBLK // (L * U), edge_body, 0)

    _start(e_hbm, ebase, EBLK, e_v0, se0)
    _start(ei_hbm, NUM_EDGES + ebase, EBLK, i_v0, si0)

    def dbl_body(g, _):
        b0 = ebase + 2 * g * EBLK
        b1 = b0 + EBLK
        _start(e_hbm, b1, EBLK, e_v1, se1)
        _start(ei_hbm, NUM_EDGES + b1, EBLK, i_v1, si1)
        _wait(e_hbm, b0, EBLK, e_v0, se0)
        _wait(ei_hbm, b0, EBLK, i_v0, si0)

        @pl.when(g > 0)
        def _():
            pltpu.make_async_copy(o_v0, alpha_hbm.at[pl.ds(b0, EBLK)],
                                  so0).wait()

        gath_chunk(e_v0, i_v0, o_v0)
        pltpu.make_async_copy(o_v0, alpha_hbm.at[pl.ds(b0, EBLK)],
                              so0).start()

        @pl.when(g + 1 < NBLK // 2)
        def _():
            b2 = b0 + 2 * EBLK
            _start(e_hbm, b2, EBLK, e_v0, se0)
            _start(ei_hbm, NUM_EDGES + b2, EBLK, i_v0, si0)

        _wait(e_hbm, b1, EBLK, e_v1, se1)
        _wait(ei_hbm, b1, EBLK, i_v1, si1)

        @pl.when(g > 0)
        def _():
            pltpu.make_async_copy(o_v1, alpha_hbm.at[pl.ds(b1, EBLK)],
                                  so1).wait()

        gath_chunk(e_v1, i_v1, o_v1)
        pltpu.make_async_copy(o_v1, alpha_hbm.at[pl.ds(b1, EBLK)],
                              so1).start()
        return 0

    lax.fori_loop(0, NBLK // 2, dbl_body, 0)
    last = ebase + (NBLK - 2) * EBLK
    pltpu.make_async_copy(o_v0, alpha_hbm.at[pl.ds(last, EBLK)], so0).wait()
    pltpu.make_async_copy(o_v1, alpha_hbm.at[pl.ds(last + EBLK, EBLK)],
                          so1).wait()


@functools.lru_cache(maxsize=None)
def _build():
    phase1 = pl.kernel(
        _phase1,
        out_type=jax.ShapeDtypeStruct((NC, NPAD), jnp.float32),
        mesh=_mesh(),
        compiler_params=pltpu.CompilerParams(needs_layout_passes=False),
        scratch_types=[
            pltpu.VMEM((NPAD,), jnp.float32),
            pltpu.VMEM((EBLK,), jnp.float32),
            pltpu.VMEM((EBLK,), jnp.float32),
            pltpu.VMEM((EBLK,), jnp.int32),
            pltpu.VMEM((EBLK,), jnp.int32),
            pltpu.VMEM_SHARED((NS, NSLICE), jnp.float32),
            pltpu.SemaphoreType.DMA,
            pltpu.SemaphoreType.DMA,
            pltpu.SemaphoreType.DMA,
            pltpu.SemaphoreType.DMA,
        ],
    )
    phase2 = pl.kernel(
        _phase2,
        out_type=jax.ShapeDtypeStruct((NUM_EDGES,), jnp.float32),
        mesh=_mesh(),
        compiler_params=pltpu.CompilerParams(needs_layout_passes=False),
        scratch_types=[
            pltpu.VMEM((NPAD,), jnp.float32),
            pltpu.VMEM((EBLK,), jnp.float32),
            pltpu.VMEM((EBLK,), jnp.float32),
            pltpu.VMEM((EBLK,), jnp.int32),
            pltpu.VMEM((EBLK,), jnp.int32),
            pltpu.VMEM((EBLK,), jnp.float32),
            pltpu.VMEM((EBLK,), jnp.float32),
            pltpu.SemaphoreType.DMA,
            pltpu.SemaphoreType.DMA,
            pltpu.SemaphoreType.DMA,
            pltpu.SemaphoreType.DMA,
            pltpu.SemaphoreType.DMA,
            pltpu.SemaphoreType.DMA,
        ],
    )
    return phase1, phase2


@jax.jit
def kernel(e, edge_index):
    phase1, phase2 = _build()
    dst_flat = edge_index.astype(jnp.int32).reshape(-1)
    partials = phase1(e, dst_flat)
    return phase2(e, dst_flat, partials)


# speedup vs baseline: 325.5434x; 2.3075x over previous
import functools

import jax
import jax.numpy as jnp
from jax import lax
from jax.experimental import pallas as pl
from jax.experimental.pallas import tpu as pltpu
from jax.experimental.pallas import tpu_sc as plsc

NUM_NODES = 100000
NUM_EDGES = 6400000

NC = 2
NS = 16
NW = NC * NS
L = 16

EPT = NUM_EDGES // NW
EBLK = 4000
NBLK = EPT // EBLK
U = 5
NPAD = 102400
NSLICE = NPAD // NS
CCH = 3200


def _mesh():
    return plsc.VectorSubcoreMesh(
        core_axis_name="c",
        subcore_axis_name="s",
        num_cores=NC,
        num_subcores=NS,
    )


def _start(hbm, off, n, buf, sem):
    pltpu.make_async_copy(hbm.at[pl.ds(off, n)], buf, sem).start()


def _wait(hbm, off, n, buf, sem):
    pltpu.make_async_copy(hbm.at[pl.ds(off, n)], buf, sem).wait()


def _phase1(e_hbm, ei_hbm, part_hbm, table_v, e_v0, e_v1, i_v0, i_v1,
            shared_v, se0, se1, si0, si1):
    cid = lax.axis_index("c")
    sid = lax.axis_index("s")
    wid = cid * NS + sid
    ebase = wid * EPT

    @plsc.parallel_loop(0, NPAD // L, unroll=8)
    def zero_body(i):
        table_v[pl.ds(i * L, L)] = jnp.zeros((L,), jnp.float32)

    def scat_chunk(eb, ib):
        @plsc.parallel_loop(0, EBLK // L, unroll=U)
        def edge_body(i):
            s = pl.ds(i * L, L)
            plsc.addupdate_scatter(table_v, [ib[s]], jnp.exp(eb[s]))

    _start(e_hbm, ebase, EBLK, e_v0, se0)
    _start(ei_hbm, NUM_EDGES + ebase, EBLK, i_v0, si0)

    def dbl_body(g, _):
        b1 = ebase + (2 * g + 1) * EBLK
        _start(e_hbm, b1, EBLK, e_v1, se1)
        _start(ei_hbm, NUM_EDGES + b1, EBLK, i_v1, si1)
        _wait(e_hbm, b1, EBLK, e_v0, se0)
        _wait(ei_hbm, b1, EBLK, i_v0, si0)
        scat_chunk(e_v0, i_v0)

        @pl.when(g + 1 < NBLK // 2)
        def _():
            b2 = ebase + (2 * g + 2) * EBLK
            _start(e_hbm, b2, EBLK, e_v0, se0)
            _start(ei_hbm, NUM_EDGES + b2, EBLK, i_v0, si0)

        _wait(e_hbm, b1, EBLK, e_v1, se1)
        _wait(ei_hbm, b1, EBLK, i_v1, si1)
        scat_chunk(e_v1, i_v1)
        return 0

    lax.fori_loop(0, NBLK // 2, dbl_body, 0)

    H = NSLICE // 2

    def add_half(h, buf):
        @plsc.parallel_loop(0, H // L, unroll=8)
        def add_body(i):
            s = pl.ds(sid * NSLICE + h * H + i * L, L)
            table_v[s] = table_v[s] + buf[pl.ds(i * L, L)]

    def round_body(r, _):
        o = lax.rem(sid + r, NS)
        pltpu.sync_copy(table_v.at[pl.ds(o * NSLICE, NSLICE)], shared_v.at[o])
        plsc.subcore_barrier()
        pltpu.sync_copy(shared_v.at[sid, pl.ds(0, H)], e_v0.at[pl.ds(0, H)])
        pltpu.sync_copy(shared_v.at[sid, pl.ds(H, H)], e_v1.at[pl.ds(0, H)])
        add_half(0, e_v0)
        add_half(1, e_v1)
        plsc.subcore_barrier()
        return 0

    lax.fori_loop(1, NS, round_body, 0)
    pltpu.sync_copy(table_v.at[pl.ds(sid * NSLICE, NSLICE)],
                    part_hbm.at[cid, pl.ds(sid * NSLICE, NSLICE)])


def _phase2(e_hbm, ei_hbm, part_hbm, alpha_hbm, table_v, e_v0, e_v1, i_v0,
            i_v1, o_v0, o_v1, se0, se1, si0, si1, so0, so1):
    cid = lax.axis_index("c")
    sid = lax.axis_index("s")
    wid = cid * NS + sid
    ebase = wid * EPT

    pltpu.sync_copy(part_hbm.at[0], table_v)

    def comb_chunk(c, buf):
        @plsc.parallel_loop(0, CCH // L, unroll=8)
        def add_body(i):
            s = pl.ds(c * CCH + i * L, L)
            table_v[s] = table_v[s] + buf[pl.ds(i * L, L)]

    def _startp(c, buf, sem):
        pltpu.make_async_copy(part_hbm.at[1, pl.ds(c * CCH, CCH)],
                              buf.at[pl.ds(0, CCH)], sem).start()

    def _waitp(buf, sem):
        pltpu.make_async_copy(part_hbm.at[1, pl.ds(0, CCH)],
                              buf.at[pl.ds(0, CCH)], sem).wait()

    NCH = NPAD // CCH
    _startp(0, e_v0, se0)

    def comb_body(g, _):
        c0 = 2 * g
        _startp(c0 + 1, e_v1, se1)
        _waitp(e_v0, se0)
        comb_chunk(c0, e_v0)

        @pl.when(c0 + 2 < NCH)
        def _():
            _startp(c0 + 2, e_v0, se0)

        _waitp(e_v1, se1)
        comb_chunk(c0 + 1, e_v1)
        return 0

    lax.fori_loop(0, NCH // 2, comb_body, 0)

    def gath_chunk(eb, ib, ob):
        @plsc.parallel_loop(0, EBLK // L, unroll=U)
        def edge_body(i):
            s = pl.ds(i * L, L)
            g = plsc.load_gather(table_v, [ib[s]])
            ob[s] = jnp.exp(eb[s]) / (g + 1e-16)

    _start(e_hbm, ebase, EBLK, e_v0, se0)
    _start(ei_hbm, NUM_EDGES + ebase, EBLK, i_v0, si0)

    def dbl_body(g, _):
        b0 = ebase + 2 * g * EBLK
        b1 = b0 + EBLK
        _start(e_hbm, b1, EBLK, e_v1, se1)
        _start(ei_hbm, NUM_EDGES + b1, EBLK, i_v1, si1)
        _wait(e_hbm, b0, EBLK, e_v0, se0)
        _wait(ei_hbm, b0, EBLK, i_v0, si0)

        @pl.when(g > 0)
        def _():
            pltpu.make_async_copy(o_v0, alpha_hbm.at[pl.ds(b0, EBLK)],
                                  so0).wait()

        gath_chunk(e_v0, i_v0, o_v0)
        pltpu.make_async_copy(o_v0, alpha_hbm.at[pl.ds(b0, EBLK)],
                              so0).start()

        @pl.when(g + 1 < NBLK // 2)
        def _():
            b2 = b0 + 2 * EBLK
            _start(e_hbm, b2, EBLK, e_v0, se0)
            _start(ei_hbm, NUM_EDGES + b2, EBLK, i_v0, si0)

        _wait(e_hbm, b1, EBLK, e_v1, se1)
        _wait(ei_hbm, b1, EBLK, i_v1, si1)

        @pl.when(g > 0)
        def _():
            pltpu.make_async_copy(o_v1, alpha_hbm.at[pl.ds(b1, EBLK)],
                                  so1).wait()

        gath_chunk(e_v1, i_v1, o_v1)
        pltpu.make_async_copy(o_v1, alpha_hbm.at[pl.ds(b1, EBLK)],
                              so1).start()
        return 0

    lax.fori_loop(0, NBLK // 2, dbl_body, 0)
    last = ebase + (NBLK - 2) * EBLK
    pltpu.make_async_copy(o_v0, alpha_hbm.at[pl.ds(last, EBLK)], so0).wait()
    pltpu.make_async_copy(o_v1, alpha_hbm.at[pl.ds(last + EBLK, EBLK)],
                          so1).wait()


@functools.lru_cache(maxsize=None)
def _build():
    phase1 = pl.kernel(
        _phase1,
        out_type=jax.ShapeDtypeStruct((NC, NPAD), jnp.float32),
        mesh=_mesh(),
        compiler_params=pltpu.CompilerParams(needs_layout_passes=False),
        scratch_types=[
            pltpu.VMEM((NPAD,), jnp.float32),
            pltpu.VMEM((EBLK,), jnp.float32),
            pltpu.VMEM((EBLK,), jnp.float32),
            pltpu.VMEM((EBLK,), jnp.int32),
            pltpu.VMEM((EBLK,), jnp.int32),
            pltpu.VMEM_SHARED((NS, NSLICE), jnp.float32),
            pltpu.SemaphoreType.DMA,
            pltpu.SemaphoreType.DMA,
            pltpu.SemaphoreType.DMA,
            pltpu.SemaphoreType.DMA,
        ],
    )
    phase2 = pl.kernel(
        _phase2,
        out_type=jax.ShapeDtypeStruct((NUM_EDGES,), jnp.float32),
        mesh=_mesh(),
        compiler_params=pltpu.CompilerParams(needs_layout_passes=False),
        scratch_types=[
            pltpu.VMEM((NPAD,), jnp.float32),
            pltpu.VMEM((EBLK,), jnp.float32),
            pltpu.VMEM((EBLK,), jnp.float32),
            pltpu.VMEM((EBLK,), jnp.int32),
            pltpu.VMEM((EBLK,), jnp.int32),
            pltpu.VMEM((EBLK,), jnp.float32),
            pltpu.VMEM((EBLK,), jnp.float32),
            pltpu.SemaphoreType.DMA,
            pltpu.SemaphoreType.DMA,
            pltpu.SemaphoreType.DMA,
            pltpu.SemaphoreType.DMA,
            pltpu.SemaphoreType.DMA,
            pltpu.SemaphoreType.DMA,
        ],
    )
    return phase1, phase2


@jax.jit
def kernel(e, edge_index):
    phase1, phase2 = _build()
    dst_flat = edge_index.astype(jnp.int32).reshape(-1)
    partials = phase1(e, dst_flat)
    return phase2(e, dst_flat, partials)


# speedup vs baseline: 327.8067x; 1.0070x over previous
import functools

import jax
import jax.numpy as jnp
from jax import lax
from jax.experimental import pallas as pl
from jax.experimental.pallas import tpu as pltpu
from jax.experimental.pallas import tpu_sc as plsc

NUM_NODES = 100000
NUM_EDGES = 6400000

NC = 2
NS = 16
NW = NC * NS
L = 16

EPT = NUM_EDGES // NW
EB1 = 2000
NB1 = EPT // EB1
EB2 = 4000
NB2 = EPT // EB2
U = 5
NPAD = 102400
NSLICE = NPAD // NS
H = NSLICE // 2
CCH = 3200


def _mesh():
    return plsc.VectorSubcoreMesh(
        core_axis_name="c",
        subcore_axis_name="s",
        num_cores=NC,
        num_subcores=NS,
    )


def _start(hbm, off, n, buf, sem):
    pltpu.make_async_copy(hbm.at[pl.ds(off, n)], buf, sem).start()


def _wait(hbm, off, n, buf, sem):
    pltpu.make_async_copy(hbm.at[pl.ds(off, n)], buf, sem).wait()


def _phase1(e_hbm, ei_hbm, part_hbm, table_v, e_v0, e_v1, i_v0, i_v1, h_v0,
            h_v1, shared_v, se0, se1, si0, si1, sh0, sh1):
    cid = lax.axis_index("c")
    sid = lax.axis_index("s")
    wid = cid * NS + sid
    ebase = wid * EPT

    _start(e_hbm, ebase, EB1, e_v0, se0)
    _start(ei_hbm, ebase, EB1, i_v0, si0)

    @plsc.parallel_loop(0, NPAD // L, unroll=8)
    def zero_body(i):
        table_v[pl.ds(i * L, L)] = jnp.zeros((L,), jnp.float32)

    def scat_chunk(eb, ib):
        @plsc.parallel_loop(0, EB1 // L, unroll=U)
        def edge_body(i):
            s = pl.ds(i * L, L)
            plsc.addupdate_scatter(table_v, [ib[s]], jnp.exp(eb[s]))

    def dbl_body(g, _):
        b1 = ebase + (2 * g + 1) * EB1
        _start(e_hbm, b1, EB1, e_v1, se1)
        _start(ei_hbm, b1, EB1, i_v1, si1)
        _wait(e_hbm, b1, EB1, e_v0, se0)
        _wait(ei_hbm, b1, EB1, i_v0, si0)
        scat_chunk(e_v0, i_v0)

        @pl.when(g + 1 < NB1 // 2)
        def _():
            b2 = ebase + (2 * g + 2) * EB1
            _start(e_hbm, b2, EB1, e_v0, se0)
            _start(ei_hbm, b2, EB1, i_v0, si0)

        _wait(e_hbm, b1, EB1, e_v1, se1)
        _wait(ei_hbm, b1, EB1, i_v1, si1)
        scat_chunk(e_v1, i_v1)
        return 0

    lax.fori_loop(0, NB1 // 2, dbl_body, 0)

    def publish(r):
        o = lax.rem(sid + r, NS)
        pltpu.sync_copy(table_v.at[pl.ds(o * NSLICE, NSLICE)],
                        shared_v.at[lax.rem(r, 2), o])

    def add_half(h, buf):
        @plsc.parallel_loop(0, H // L, unroll=8)
        def add_body(i):
            s = pl.ds(sid * NSLICE + h * H + i * L, L)
            table_v[s] = table_v[s] + buf[pl.ds(i * L, L)]

    publish(1)

    def round_body(r, _):
        plsc.subcore_barrier()
        p = lax.rem(r, 2)
        c0 = pltpu.make_async_copy(shared_v.at[p, sid, pl.ds(0, H)], h_v0,
                                   sh0)
        c1 = pltpu.make_async_copy(shared_v.at[p, sid, pl.ds(H, H)], h_v1,
                                   sh1)
        c0.start()
        c1.start()

        @pl.when(r + 1 < NS)
        def _():
            publish(r + 1)

        c0.wait()
        add_half(0, h_v0)
        c1.wait()
        add_half(1, h_v1)
        return 0

    lax.fori_loop(1, NS, round_body, 0)
    pltpu.sync_copy(table_v.at[pl.ds(sid * NSLICE, NSLICE)],
                    part_hbm.at[cid, pl.ds(sid * NSLICE, NSLICE)])


def _phase2(e_hbm, ei_hbm, part_hbm, alpha_hbm, table_v, e_v0, e_v1, i_v0,
            i_v1, o_v0, o_v1, se0, se1, si0, si1, so0, so1):
    cid = lax.axis_index("c")
    sid = lax.axis_index("s")
    wid = cid * NS + sid
    ebase = wid * EPT

    _start(e_hbm, ebase, EB2, e_v0, se0)
    _start(ei_hbm, ebase, EB2, i_v0, si0)

    p0 = pltpu.make_async_copy(part_hbm.at[0], table_v, se1)
    p0.start()

    def comb_chunk(c, buf):
        @plsc.parallel_loop(0, CCH // L, unroll=8)
        def add_body(i):
            s = pl.ds(c * CCH + i * L, L)
            table_v[s] = table_v[s] + buf[pl.ds(i * L, L)]

    def _startp(c, buf, sem):
        pltpu.make_async_copy(part_hbm.at[1, pl.ds(c * CCH, CCH)],
                              buf.at[pl.ds(0, CCH)], sem).start()

    def _waitp(buf, sem):
        pltpu.make_async_copy(part_hbm.at[1, pl.ds(0, CCH)],
                              buf.at[pl.ds(0, CCH)], sem).wait()

    NCH = NPAD // CCH
    _startp(0, o_v0, so0)
    _startp(1, o_v1, so1)
    p0.wait()

    def comb_body(g, _):
        c0 = 2 * g
        _waitp(o_v0, so0)
        comb_chunk(c0, o_v0)

        @pl.when(c0 + 2 < NCH)
        def _():
            _startp(c0 + 2, o_v0, so0)

        _waitp(o_v1, so1)
        comb_chunk(c0 + 1, o_v1)

        @pl.when(c0 + 3 < NCH)
        def _():
            _startp(c0 + 3, o_v1, so1)

        return 0

    lax.fori_loop(0, NCH // 2, comb_body, 0)

    def gath_chunk(eb, ib, ob):
        @plsc.parallel_loop(0, EB2 // L, unroll=U)
        def edge_body(i):
            s = pl.ds(i * L, L)
            g = plsc.load_gather(table_v, [ib[s]])
            ob[s] = jnp.exp(eb[s]) / (g + 1e-16)

    def dbl_body(g, _):
        b0 = ebase + 2 * g * EB2
        b1 = b0 + EB2
        _start(e_hbm, b1, EB2, e_v1, se1)
        _start(ei_hbm, b1, EB2, i_v1, si1)
        _wait(e_hbm, b0, EB2, e_v0, se0)
        _wait(ei_hbm, b0, EB2, i_v0, si0)

        @pl.when(g > 0)
        def _():
            pltpu.make_async_copy(o_v0, alpha_hbm.at[pl.ds(b0, EB2)],
                                  so0).wait()

        gath_chunk(e_v0, i_v0, o_v0)
        pltpu.make_async_copy(o_v0, alpha_hbm.at[pl.ds(b0, EB2)],
                              so0).start()

        @pl.when(g + 1 < NB2 // 2)
        def _():
            b2 = b0 + 2 * EB2
            _start(e_hbm, b2, EB2, e_v0, se0)
            _start(ei_hbm, b2, EB2, i_v0, si0)

        _wait(e_hbm, b1, EB2, e_v1, se1)
        _wait(ei_hbm, b1, EB2, i_v1, si1)

        @pl.when(g > 0)
        def _():
            pltpu.make_async_copy(o_v1, alpha_hbm.at[pl.ds(b1, EB2)],
                                  so1).wait()

        gath_chunk(e_v1, i_v1, o_v1)
        pltpu.make_async_copy(o_v1, alpha_hbm.at[pl.ds(b1, EB2)],
                              so1).start()
        return 0

    lax.fori_loop(0, NB2 // 2, dbl_body, 0)
    last = ebase + (NB2 - 2) * EB2
    pltpu.make_async_copy(o_v0, alpha_hbm.at[pl.ds(last, EB2)], so0).wait()
    pltpu.make_async_copy(o_v1, alpha_hbm.at[pl.ds(last + EB2, EB2)],
                          so1).wait()


@functools.lru_cache(maxsize=None)
def _build():
    phase1 = pl.kernel(
        _phase1,
        out_type=jax.ShapeDtypeStruct((NC, NPAD), jnp.float32),
        mesh=_mesh(),
        compiler_params=pltpu.CompilerParams(needs_layout_passes=False),
        scratch_types=[
            pltpu.VMEM((NPAD,), jnp.float32),
            pltpu.VMEM((EB1,), jnp.float32),
            pltpu.VMEM((EB1,), jnp.float32),
            pltpu.VMEM((EB1,), jnp.int32),
            pltpu.VMEM((EB1,), jnp.int32),
            pltpu.VMEM((H,), jnp.float32),
            pltpu.VMEM((H,), jnp.float32),
            pltpu.VMEM_SHARED((2, NS, NSLICE), jnp.float32),
            pltpu.SemaphoreType.DMA,
            pltpu.SemaphoreType.DMA,
            pltpu.SemaphoreType.DMA,
            pltpu.SemaphoreType.DMA,
            pltpu.SemaphoreType.DMA,
            pltpu.SemaphoreType.DMA,
        ],
    )
    phase2 = pl.kernel(
        _phase2,
        out_type=jax.ShapeDtypeStruct((NUM_EDGES,), jnp.float32),
        mesh=_mesh(),
        compiler_params=pltpu.CompilerParams(needs_layout_passes=False),
        scratch_types=[
            pltpu.VMEM((NPAD,), jnp.float32),
            pltpu.VMEM((EB2,), jnp.float32),
            pltpu.VMEM((EB2,), jnp.float32),
            pltpu.VMEM((EB2,), jnp.int32),
            pltpu.VMEM((EB2,), jnp.int32),
            pltpu.VMEM((EB2,), jnp.float32),
            pltpu.VMEM((EB2,), jnp.float32),
            pltpu.SemaphoreType.DMA,
            pltpu.SemaphoreType.DMA,
            pltpu.SemaphoreType.DMA,
            pltpu.SemaphoreType.DMA,
            pltpu.SemaphoreType.DMA,
            pltpu.SemaphoreType.DMA,
        ],
    )
    return phase1, phase2


@jax.jit
def kernel(e, edge_index):
    phase1, phase2 = _build()
    dst = edge_index[1].astype(jnp.int32)
    partials = phase1(e, dst)
    return phase2(e, dst, partials)
